# Initial kernel scaffold; baseline (speedup 1.0000x reference)
import jax
import jax.numpy as jnp
from jax.experimental import pallas as pl


def kernel(hm_out, wh_out, hps_out, dim_out, rot_out, reg_out, hm_hp_out, hp_offset_out, p3d_out, prob_out, hm_gt, hps_gt, hps_mask, dep, reg_mask, wh_gt, dim_gt, p3d_gt, rot_mask, rotres, reg_gt, hp_mask, hp_offset_gt, hm_hp_gt, ind, hp_ind, rotbin):
    raise NotImplementedError("write your pallas kernel here")


# speedup vs baseline: 1.0188x; 1.0188x over previous
import functools

import jax
import jax.numpy as jnp
from jax import lax
from jax.experimental import pallas as pl
from jax.experimental.pallas import tpu as pltpu
from jax.experimental.pallas import tpu_sc as plsc

B = 8
H = 96
W = 320
K = 32
HW = H * W

_MAPS = (
    ("hps", 32, False),
    ("p3d", 48, False),
    ("rot", 8, False),
    ("wh", 2, False),
    ("dim", 3, False),
    ("reg", 2, False),
    ("prob", 1, False),
    ("hpo", 2, True),
)
_NTILES = 32
_COUNTS = tuple((B * (K * 16 if hp else K) * c) for _, c, hp in _MAPS)
_CHUNKS = tuple(n // _NTILES for n in _COUNTS)
_TILE_ELEMS = sum(_CHUNKS)
_TOTAL = _TILE_ELEMS * _NTILES
_LOFFS = tuple(sum(_CHUNKS[:i]) for i in range(len(_CHUNKS)))


def _sc_gather_kernel(idx_hbm, hps_t, p3d_t, rot_t, wh_t, dim_t, reg_t,
                      prob_t, hpo_t, out_hbm, idx_v, data_v, sem):
    tables = (hps_t, p3d_t, rot_t, wh_t, dim_t, reg_t, prob_t, hpo_t)
    wid = lax.axis_index("c") * 16 + lax.axis_index("s")
    base = wid * _TILE_ELEMS
    pltpu.sync_copy(idx_hbm.at[pl.ds(base, _TILE_ELEMS)], idx_v)
    copies = []
    for table, loff, chunk in zip(tables, _LOFFS, _CHUNKS):
        for sub in range(0, chunk, 128):
            n = min(128, chunk - sub)
            o = loff + sub
            copies.append(pltpu.make_async_copy(
                table.at[idx_v.at[pl.ds(o, n)]], data_v.at[pl.ds(o, n)], sem))
    for cp in copies:
        cp.start()
    for cp in copies:
        cp.wait()
    pltpu.sync_copy(data_v, out_hbm.at[pl.ds(base, _TILE_ELEMS)])


@jax.jit
def _sc_gather(idx_all, hps_t, p3d_t, rot_t, wh_t, dim_t, reg_t, prob_t,
               hpo_t):
    run = pl.kernel(
        _sc_gather_kernel,
        out_type=jax.ShapeDtypeStruct((_TOTAL,), jnp.float32),
        mesh=plsc.VectorSubcoreMesh(core_axis_name="c", subcore_axis_name="s"),
        scratch_types=[
            pltpu.VMEM((_TILE_ELEMS,), jnp.int32),
            pltpu.VMEM((_TILE_ELEMS,), jnp.float32),
            pltpu.SemaphoreType.DMA,
        ],
    )
    return run(idx_all, hps_t, p3d_t, rot_t, wh_t, dim_t, reg_t, prob_t,
               hpo_t)


_HM_ROWS = B * 3 * HW // 1280
_HP_ROWS = B * 16 * HW // 1280
_FGRID = 8


def _focal_terms(x, gt):
    pred = jnp.clip(1.0 / (1.0 + jnp.exp(-x)), 1e-4, 1.0 - 1e-4)
    pos = (gt == 1.0).astype(jnp.float32)
    neg = (gt < 1.0).astype(jnp.float32)
    om = 1.0 - pred
    omg = 1.0 - gt
    pls = jnp.sum(jnp.log(pred) * om * om * pos)
    nls = jnp.sum(jnp.log(om) * pred * pred * (omg * omg) * (omg * omg) * neg)
    return pls, nls, jnp.sum(pos)


def _focal_body(hm_ref, hmgt_ref, hp_ref, hpgt_ref, out_ref):
    i = pl.program_id(0)

    @pl.when(i == 0)
    def _():
        out_ref[...] = jnp.zeros_like(out_ref)

    a = _focal_terms(hm_ref[...], hmgt_ref[...])
    b = _focal_terms(hp_ref[...], hpgt_ref[...])
    r = lax.broadcasted_iota(jnp.int32, (8, 128), 0)
    c = lax.broadcasted_iota(jnp.int32, (8, 128), 1)
    contrib = jnp.zeros((8, 128), jnp.float32)
    for row, vals in ((0, a), (1, b)):
        for col, v in enumerate(vals):
            contrib = contrib + jnp.where((r == row) & (c == col), v, 0.0)
    out_ref[...] += contrib


def _focal_partials(hm2, hmgt2, hp2, hpgt2):
    hm_blk = _HM_ROWS // _FGRID
    hp_blk = _HP_ROWS // _FGRID
    return pl.pallas_call(
        _focal_body,
        grid=(_FGRID,),
        in_specs=[
            pl.BlockSpec((hm_blk, 1280), lambda i: (i, 0)),
            pl.BlockSpec((hm_blk, 1280), lambda i: (i, 0)),
            pl.BlockSpec((hp_blk, 1280), lambda i: (i, 0)),
            pl.BlockSpec((hp_blk, 1280), lambda i: (i, 0)),
        ],
        out_specs=pl.BlockSpec((8, 128), lambda i: (0, 0)),
        out_shape=jax.ShapeDtypeStruct((8, 128), jnp.float32),
    )(hm2, hmgt2, hp2, hpgt2)


def _sig(x):
    return jnp.clip(1.0 / (1.0 + jnp.exp(-x)), 1e-4, 1.0 - 1e-4)


def _masked_l1(pred, gt, m):
    return jnp.sum(jnp.abs(pred * m - gt * m)) / (jnp.sum(m) + 1e-4)


def _sl1(a, b):
    d = jnp.abs(a - b)
    return jnp.where(d < 1.0, 0.5 * d * d, d - 0.5)


def _bin_loss(logits, target, mask):
    lo = logits * mask
    mx = jnp.max(lo, axis=-1, keepdims=True)
    ls = lo - mx - jnp.log(jnp.sum(jnp.exp(lo - mx), axis=-1, keepdims=True))
    ce = -jnp.where(target == 0, ls[:, 0], ls[:, 1])
    return jnp.mean(ce)


def _final_body(fp_ref, hps_p, hps_g, hps_m, dep_ref, regm_ref, wh_p, wh_g,
                dim_p, dim_g, p3d_p, p3d_g, rot_p, rotb, rotr, rotm, reg_p,
                reg_g, prob_p, hpo_p, hpo_g, hpm_ref, out_ref):
    fp = fp_ref[...]

    def focal_final(row):
        pls, nls, npos = fp[row, 0], fp[row, 1], fp[row, 2]
        return jnp.where(npos > 0.0,
                         -(pls + nls) / jnp.maximum(npos, 1.0), -nls)

    hm_loss = focal_final(0)
    hm_hp_loss = focal_final(1)

    hm3 = hps_m[...]
    dep = jnp.maximum(dep_ref[...], 1.0)
    kps = hps_p[...]
    hgt = hps_g[...]
    hp_loss = jnp.sum(jnp.abs(kps * hm3 - hgt * hm3) / dep) / (
        jnp.sum(hm3) + 1e-4)
    coor_loss = _masked_l1(kps, hgt, hm3)

    rm = regm_ref[...]
    rm2 = jnp.broadcast_to(rm, (B * K, 2))
    rm3 = jnp.broadcast_to(rm, (B * K, 3))
    rm48 = jnp.broadcast_to(rm, (B * K, 48))
    wh_loss = _masked_l1(wh_p[...], wh_g[...], rm2)
    dim_loss = _masked_l1(dim_p[...], dim_g[...], rm3)
    p3d_loss = _masked_l1(p3d_p[...], p3d_g[...], rm48)
    off_loss = _masked_l1(reg_p[...], reg_g[...], rm2)

    hpm2 = jnp.broadcast_to(hpm_ref[...], (B * K * 16, 2))
    hp_offset_loss = _masked_l1(hpo_p[...], hpo_g[...], hpm2)

    rp = rot_p[...]
    tb = rotb[...]
    tr = rotr[...]
    m1 = rotm[...]
    lb1 = _bin_loss(rp[:, 0:2], tb[:, 0], m1)
    lb2 = _bin_loss(rp[:, 4:6], tb[:, 1], m1)
    w1 = (tb[:, 0] == 1).astype(jnp.float32)
    w2 = (tb[:, 1] == 1).astype(jnp.float32)
    ls1 = jnp.sum(_sl1(rp[:, 2], jnp.sin(tr[:, 0])) * w1) / (jnp.sum(w1) + 1e-4)
    lc1 = jnp.sum(_sl1(rp[:, 3], jnp.cos(tr[:, 0])) * w1) / (jnp.sum(w1) + 1e-4)
    ls2 = jnp.sum(_sl1(rp[:, 6], jnp.sin(tr[:, 1])) * w2) / (jnp.sum(w2) + 1e-4)
    lc2 = jnp.sum(_sl1(rp[:, 7], jnp.cos(tr[:, 1])) * w2) / (jnp.sum(w2) + 1e-4)
    rot_loss = lb1 + lb2 + ls1 + lc1 + ls2 + lc2

    prob = _sig(prob_p[...])
    tgt = jnp.exp(-coor_loss)
    prob_loss = jnp.mean(jnp.abs(prob - tgt))
    box_score = coor_loss + prob_loss

    vals = (box_score, hm_loss, hp_loss, hm_hp_loss, hp_offset_loss, wh_loss,
            off_loss, dim_loss, rot_loss, prob_loss, box_score, coor_loss,
            p3d_loss)
    col = lax.broadcasted_iota(jnp.int32, (1, 16), 1)
    acc = jnp.zeros((1, 16), jnp.float32)
    for j, v in enumerate(vals):
        acc = acc + jnp.where(col == j, v, 0.0)
    out_ref[...] = acc


def _finalize(*args):
    return pl.pallas_call(
        _final_body,
        out_shape=jax.ShapeDtypeStruct((1, 16), jnp.float32),
    )(*args)


def kernel(hm_out, wh_out, hps_out, dim_out, rot_out, reg_out, hm_hp_out,
           hp_offset_out, p3d_out, prob_out, hm_gt, hps_gt, hps_mask, dep,
           reg_mask, wh_gt, dim_gt, p3d_gt, rot_mask, rotres, reg_gt, hp_mask,
           hp_offset_gt, hm_hp_gt, ind, hp_ind, rotbin):
    ind32 = ind.astype(jnp.int32)
    hp_ind32 = hp_ind.astype(jnp.int32)

    idx_parts = []
    for (_, c, use_hp), chunk in zip(_MAPS, _CHUNKS):
        src = hp_ind32 if use_hp else ind32
        bcol = jnp.arange(B, dtype=jnp.int32)[:, None, None] * (c * HW)
        ccol = jnp.arange(c, dtype=jnp.int32)[None, None, :] * HW
        flat = (bcol + ccol + src[:, :, None]).reshape(_NTILES, chunk)
        idx_parts.append(flat)
    idx_all = jnp.concatenate(idx_parts, axis=1).reshape(-1)

    gathered = _sc_gather(
        idx_all,
        hps_out.reshape(-1), p3d_out.reshape(-1), rot_out.reshape(-1),
        wh_out.reshape(-1), dim_out.reshape(-1), reg_out.reshape(-1),
        prob_out.reshape(-1), hp_offset_out.reshape(-1))
    g = gathered.reshape(_NTILES, _TILE_ELEMS)
    preds = {}
    for (name, c, use_hp), loff, chunk in zip(_MAPS, _LOFFS, _CHUNKS):
        rows = B * (K * 16 if use_hp else K)
        preds[name] = g[:, loff:loff + chunk].reshape(rows, c)

    fp = _focal_partials(
        hm_out.reshape(_HM_ROWS, 1280), hm_gt.reshape(_HM_ROWS, 1280),
        hm_hp_out.reshape(_HP_ROWS, 1280), hm_hp_gt.reshape(_HP_ROWS, 1280))

    out = _finalize(
        fp,
        preds["hps"], hps_gt.reshape(B * K, 32), hps_mask.reshape(B * K, 32),
        dep.reshape(B * K, 1), reg_mask.reshape(B * K, 1),
        preds["wh"], wh_gt.reshape(B * K, 2),
        preds["dim"], dim_gt.reshape(B * K, 3),
        preds["p3d"], p3d_gt.reshape(B * K, 48),
        preds["rot"], rotbin.reshape(B * K, 2).astype(jnp.int32),
        rotres.reshape(B * K, 2), rot_mask.reshape(B * K, 1),
        preds["reg"], reg_gt.reshape(B * K, 2),
        preds["prob"],
        preds["hpo"], hp_offset_gt.reshape(B * K * 16, 2),
        hp_mask.reshape(B * K * 16, 1))
    return out.reshape(16)[:13]
